# baseline (speedup 1.0000x reference)
import jax
import jax.numpy as jnp
from jax import lax
from jax.experimental import pallas as pl
from jax.experimental.pallas import tpu as pltpu
from jax.experimental.pallas import tpu_sc as plsc

N = 10000
E = 320000
D = 128
NC = 2
NS = 16
NW = NC * NS
K = 64
CH = 160
EPW = K * CH
E_PAD = NW * EPW
STRIP = 640
AGG_ROWS = NS * STRIP
BN = 2000
LAST = N - (NS - 1) * STRIP
NB = 4
SS = 40
STAGES0 = 5
STAGES1 = 3


def _deg_body(dst_hbm, out_hbm, dst_v, ones_v, zero_v, deg_sh, dsem):
    c = lax.axis_index("c")
    s = lax.axis_index("s")
    w = c * NS + s

    def fill_step(i, carry):
        for j in range(D // 16):
            ones_v[i, pl.ds(j * 16, 16)] = jnp.full((16,), 1.0, jnp.float32)
            zero_v[i, pl.ds(j * 16, 16)] = jnp.zeros((16,), jnp.float32)
        return carry

    lax.fori_loop(0, K, fill_step, 0)
    for k in range(STRIP // K):
        pltpu.sync_copy(zero_v, deg_sh.at[pl.ds(s * STRIP + k * K, K)])
    plsc.subcore_barrier()

    pltpu.sync_copy(dst_hbm.at[w], dst_v)

    for j in range(4):
        pltpu.async_copy(ones_v, deg_sh.at[dst_v.at[j]], dsem, add=True)

    def step(k, carry):
        @pl.when(k + 4 < CH)
        def _issue():
            pltpu.async_copy(ones_v, deg_sh.at[dst_v.at[k + 4]], dsem,
                             add=True)

        pltpu.make_async_copy(ones_v, deg_sh.at[dst_v.at[0]], dsem).wait()
        return carry

    lax.fori_loop(0, CH, step, 0)
    plsc.subcore_barrier()

    base = s * STRIP

    @pl.when(s < NS - 1)
    def _copy_full():
        pltpu.sync_copy(deg_sh.at[pl.ds(base, STRIP)],
                        out_hbm.at[c, pl.ds(base, STRIP)])

    @pl.when(s == NS - 1)
    def _copy_tail():
        pltpu.sync_copy(deg_sh.at[pl.ds(base, LAST)],
                        out_hbm.at[c, pl.ds(base, LAST)])


def _agg_pipeline(g_hbm, src_hbm, dst_hbm, src_v, dst_v, rows_v, agg_sh,
                  sems, base, nstages):
    for stage in range(nstages):
        pltpu.sync_copy(src_hbm.at[pl.ds(base + stage * SS, SS)], src_v)
        pltpu.sync_copy(dst_hbm.at[pl.ds(base + stage * SS, SS)], dst_v)

        for j in range(NB - 1):
            pltpu.async_copy(g_hbm.at[src_v.at[j]], rows_v.at[j], sems[j])

        def outer(i, carry):
            for b in range(NB):
                k = i * NB + b
                bj = (b + NB - 1) % NB
                pltpu.make_async_copy(g_hbm.at[src_v.at[k]], rows_v.at[b],
                                      sems[b]).wait()
                pltpu.async_copy(rows_v.at[b], agg_sh.at[dst_v.at[k]],
                                 sems[b], add=True)

                @pl.when(k > 0)
                def _wait_prev_scatter():
                    pltpu.make_async_copy(rows_v.at[bj],
                                          agg_sh.at[dst_v.at[k]],
                                          sems[bj]).wait()

                @pl.when(k + NB - 1 < SS)
                def _issue_next_gather():
                    pltpu.async_copy(g_hbm.at[src_v.at[k + NB - 1]],
                                     rows_v.at[bj], sems[bj])
            return carry

        lax.fori_loop(0, SS // NB, outer, 0)
        pltpu.make_async_copy(rows_v.at[(SS - 1) % NB], agg_sh.at[dst_v.at[0]],
                              sems[(SS - 1) % NB]).wait()


def _agg_body(g_hbm, src_hbm, dst_hbm, out_hbm, src_v, dst_v, rows_v, agg_sh,
              sem0, sem1, sem2, sem3, sem4, sem5, sem6, sem7):
    sems = [sem0, sem1, sem2, sem3, sem4, sem5, sem6, sem7]
    c = lax.axis_index("c")
    s = lax.axis_index("s")

    def zero_step(i, carry):
        for j in range(D // 16):
            rows_v[0, i, pl.ds(j * 16, 16)] = jnp.zeros((16,), jnp.float32)
        return carry

    lax.fori_loop(0, K, zero_step, 0)
    for k in range(STRIP // K):
        pltpu.sync_copy(rows_v.at[0], agg_sh.at[pl.ds(s * STRIP + k * K, K)])
    plsc.subcore_barrier()

    @pl.when(c == 0)
    def _core0():
        _agg_pipeline(g_hbm, src_hbm, dst_hbm, src_v, dst_v, rows_v, agg_sh,
                      sems, s * (STAGES0 * SS), STAGES0)

    @pl.when(c == 1)
    def _core1():
        _agg_pipeline(g_hbm, src_hbm, dst_hbm, src_v, dst_v, rows_v, agg_sh,
                      sems, NS * STAGES0 * SS + s * (STAGES1 * SS), STAGES1)

    plsc.subcore_barrier()

    base = s * STRIP

    @pl.when(s < NS - 1)
    def _copy_full():
        pltpu.sync_copy(agg_sh.at[pl.ds(base, STRIP)],
                        out_hbm.at[c, pl.ds(base, STRIP)])

    @pl.when(s == NS - 1)
    def _copy_tail():
        pltpu.sync_copy(agg_sh.at[pl.ds(base, LAST)],
                        out_hbm.at[c, pl.ds(base, LAST)])


def _prologue_tc(parts_ref, x_ref, g_ref, dinv_ref):
    deg = parts_ref[0] + parts_ref[1]
    dinv = lax.rsqrt(jnp.maximum(deg, 1.0))
    g_ref[...] = dinv * x_ref[...]
    dinv_ref[...] = dinv


def _mm_tc(r0_ref, r1_ref, dinv_ref, w_ref, b_ref, h_ref, g_ref):
    u = dinv_ref[...] * (r0_ref[...] + r1_ref[...])
    h = jnp.dot(u, w_ref[...], preferred_element_type=jnp.float32) + b_ref[...]
    h_ref[...] = h
    g_ref[...] = dinv_ref[...] * h


def kernel(x, edge_index, W, b):
    assert x.shape == (N, D) and edge_index.shape == (2, E)
    num_layers = W.shape[0]
    src = edge_index[0].astype(jnp.int32)
    dst = edge_index[1].astype(jnp.int32)
    pad = E_PAD - E
    src_p = jnp.concatenate([src, jnp.zeros((pad,), jnp.int32)])
    dst_p = jnp.concatenate([dst, jnp.full((pad,), N, jnp.int32)])
    src2 = src_p.reshape(E_PAD // K, K)
    dst2 = dst_p.reshape(E_PAD // K, K)
    dst3 = dst_p.reshape(NW, CH, K)

    mesh = plsc.VectorSubcoreMesh(core_axis_name="c", subcore_axis_name="s")

    deg_call = pl.kernel(
        _deg_body,
        out_type=jax.ShapeDtypeStruct((NC, N, D), jnp.float32),
        mesh=mesh,
        scratch_types=[
            pltpu.VMEM((CH, K), jnp.int32),
            pltpu.VMEM((K, D), jnp.float32),
            pltpu.VMEM((K, D), jnp.float32),
            pltpu.VMEM_SHARED((AGG_ROWS, D), jnp.float32),
            pltpu.SemaphoreType.DMA,
        ],
    )
    parts = deg_call(dst3)

    g0, dinvb = pl.pallas_call(
        _prologue_tc,
        out_shape=[jax.ShapeDtypeStruct((N, D), jnp.float32)] * 2,
    )(parts, x)

    agg_call = pl.kernel(
        _agg_body,
        out_type=jax.ShapeDtypeStruct((NC, N, D), jnp.float32),
        mesh=mesh,
        scratch_types=[
            pltpu.VMEM((SS, K), jnp.int32),
            pltpu.VMEM((SS, K), jnp.int32),
            pltpu.VMEM((NB, K, D), jnp.float32),
            pltpu.VMEM_SHARED((AGG_ROWS, D), jnp.float32),
            pltpu.SemaphoreType.DMA,
            pltpu.SemaphoreType.DMA,
            pltpu.SemaphoreType.DMA,
            pltpu.SemaphoreType.DMA,
            pltpu.SemaphoreType.DMA,
            pltpu.SemaphoreType.DMA,
            pltpu.SemaphoreType.DMA,
            pltpu.SemaphoreType.DMA,
        ],
    )

    mm_call = pl.pallas_call(
        _mm_tc,
        grid=(N // BN,),
        in_specs=[pl.BlockSpec((BN, D), lambda i: (i, 0))] * 3 + [
            pl.BlockSpec((D, D), lambda i: (0, 0)),
            pl.BlockSpec((1, D), lambda i: (0, 0)),
        ],
        out_specs=[pl.BlockSpec((BN, D), lambda i: (i, 0))] * 2,
        out_shape=[jax.ShapeDtypeStruct((N, D), jnp.float32)] * 2,
    )

    g = g0
    outs = []
    for i in range(num_layers):
        r = agg_call(g, src2, dst2)
        h, g = mm_call(r[0], r[1], dinvb, W[i], b[i].reshape(1, D))
        outs.append(h)
    return jnp.concatenate(outs, axis=1)


# speedup vs baseline: 2.1481x; 2.1481x over previous
import jax
import jax.numpy as jnp
from jax import lax
from jax.experimental import pallas as pl
from jax.experimental.pallas import tpu as pltpu
from jax.experimental.pallas import tpu_sc as plsc

N = 10000
E = 320000
D = 128
NC = 2
NS = 16
NW = NC * NS
K = 64
CH = 160
EPW = K * CH
E_PAD = NW * EPW
STRIP = 640
AGG_ROWS = NS * STRIP
BN = 2000
LAST = N - (NS - 1) * STRIP
NB = 4
SS = 40
STAGES0 = 6
STAGES1 = 2


def _deg_body(dst_hbm, out_hbm, dst_v, ones_v, zero_v, deg_sh, dsem):
    c = lax.axis_index("c")
    s = lax.axis_index("s")
    w = c * NS + s

    def fill_step(i, carry):
        for j in range(D // 16):
            ones_v[i, pl.ds(j * 16, 16)] = jnp.full((16,), 1.0, jnp.float32)
            zero_v[i, pl.ds(j * 16, 16)] = jnp.zeros((16,), jnp.float32)
        return carry

    lax.fori_loop(0, K, fill_step, 0)
    for k in range(STRIP // K):
        pltpu.sync_copy(zero_v, deg_sh.at[pl.ds(s * STRIP + k * K, K)])
    plsc.subcore_barrier()

    pltpu.sync_copy(dst_hbm.at[w], dst_v)

    for j in range(4):
        pltpu.async_copy(ones_v, deg_sh.at[dst_v.at[j]], dsem, add=True)

    def step(k, carry):
        @pl.when(k + 4 < CH)
        def _issue():
            pltpu.async_copy(ones_v, deg_sh.at[dst_v.at[k + 4]], dsem,
                             add=True)

        pltpu.make_async_copy(ones_v, deg_sh.at[dst_v.at[0]], dsem).wait()
        return carry

    lax.fori_loop(0, CH, step, 0)
    plsc.subcore_barrier()

    base = s * STRIP

    @pl.when(s < NS - 1)
    def _copy_full():
        pltpu.sync_copy(deg_sh.at[pl.ds(base, STRIP)],
                        out_hbm.at[c, pl.ds(base, STRIP)])

    @pl.when(s == NS - 1)
    def _copy_tail():
        pltpu.sync_copy(deg_sh.at[pl.ds(base, LAST)],
                        out_hbm.at[c, pl.ds(base, LAST)])


def _agg_pipeline(g_hbm, src_hbm, dst_hbm, src_v, dst_v, rows_v, agg_sh,
                  sems, base, nstages):
    for stage in range(nstages):
        pltpu.sync_copy(src_hbm.at[pl.ds(base + stage * SS, SS)], src_v)
        pltpu.sync_copy(dst_hbm.at[pl.ds(base + stage * SS, SS)], dst_v)

        for j in range(NB - 1):
            pltpu.async_copy(g_hbm.at[src_v.at[j]], rows_v.at[j], sems[j])

        def outer(i, carry):
            for b in range(NB):
                k = i * NB + b
                bj = (b + NB - 1) % NB
                pltpu.make_async_copy(g_hbm.at[src_v.at[k]], rows_v.at[b],
                                      sems[b]).wait()
                pltpu.async_copy(rows_v.at[b], agg_sh.at[dst_v.at[k]],
                                 sems[b], add=True)

                @pl.when(k > 0)
                def _wait_prev_scatter():
                    pltpu.make_async_copy(rows_v.at[bj],
                                          agg_sh.at[dst_v.at[k]],
                                          sems[bj]).wait()

                @pl.when(k + NB - 1 < SS)
                def _issue_next_gather():
                    pltpu.async_copy(g_hbm.at[src_v.at[k + NB - 1]],
                                     rows_v.at[bj], sems[bj])
            return carry

        lax.fori_loop(0, SS // NB, outer, 0)
        pltpu.make_async_copy(rows_v.at[(SS - 1) % NB], agg_sh.at[dst_v.at[0]],
                              sems[(SS - 1) % NB]).wait()


def _agg_body(g_hbm, src_hbm, dst_hbm, out_hbm, src_v, dst_v, rows_v, agg_sh,
              sem0, sem1, sem2, sem3, sem4, sem5, sem6, sem7):
    sems = [sem0, sem1, sem2, sem3, sem4, sem5, sem6, sem7]
    c = lax.axis_index("c")
    s = lax.axis_index("s")

    def zero_step(i, carry):
        for j in range(D // 16):
            rows_v[0, i, pl.ds(j * 16, 16)] = jnp.zeros((16,), jnp.float32)
        return carry

    lax.fori_loop(0, K, zero_step, 0)
    for k in range(STRIP // K):
        pltpu.sync_copy(rows_v.at[0], agg_sh.at[pl.ds(s * STRIP + k * K, K)])
    plsc.subcore_barrier()

    @pl.when(c == 0)
    def _core0():
        _agg_pipeline(g_hbm, src_hbm, dst_hbm, src_v, dst_v, rows_v, agg_sh,
                      sems, s * (STAGES0 * SS), STAGES0)

    @pl.when(c == 1)
    def _core1():
        _agg_pipeline(g_hbm, src_hbm, dst_hbm, src_v, dst_v, rows_v, agg_sh,
                      sems, NS * STAGES0 * SS + s * (STAGES1 * SS), STAGES1)

    plsc.subcore_barrier()

    base = s * STRIP

    @pl.when(s < NS - 1)
    def _copy_full():
        pltpu.sync_copy(agg_sh.at[pl.ds(base, STRIP)],
                        out_hbm.at[c, pl.ds(base, STRIP)])

    @pl.when(s == NS - 1)
    def _copy_tail():
        pltpu.sync_copy(agg_sh.at[pl.ds(base, LAST)],
                        out_hbm.at[c, pl.ds(base, LAST)])


def _prologue_tc(parts_ref, x_ref, g_ref, dinv_ref):
    deg = parts_ref[0] + parts_ref[1]
    dinv = lax.rsqrt(jnp.maximum(deg, 1.0))
    g_ref[...] = dinv * x_ref[...]
    dinv_ref[...] = dinv


def _mm_tc(r0_ref, r1_ref, dinv_ref, w_ref, b_ref, h_ref, g_ref):
    u = dinv_ref[...] * (r0_ref[...] + r1_ref[...])
    h = jnp.dot(u, w_ref[...], preferred_element_type=jnp.float32) + b_ref[...]
    h_ref[...] = h
    g_ref[...] = dinv_ref[...] * h


def kernel(x, edge_index, W, b):
    assert x.shape == (N, D) and edge_index.shape == (2, E)
    num_layers = W.shape[0]
    src = edge_index[0].astype(jnp.int32)
    dst = edge_index[1].astype(jnp.int32)
    pad = E_PAD - E
    pi = jnp.arange(pad, dtype=jnp.int32)
    src_p = jnp.concatenate([src, pi % N])
    dst_p = jnp.concatenate([dst, N + pi % (AGG_ROWS - N)])
    src2 = src_p.reshape(E_PAD // K, K)
    dst2 = dst_p.reshape(E_PAD // K, K)
    dst3 = dst_p.reshape(NW, CH, K)

    mesh = plsc.VectorSubcoreMesh(core_axis_name="c", subcore_axis_name="s")

    deg_call = pl.kernel(
        _deg_body,
        out_type=jax.ShapeDtypeStruct((NC, N, D), jnp.float32),
        mesh=mesh,
        scratch_types=[
            pltpu.VMEM((CH, K), jnp.int32),
            pltpu.VMEM((K, D), jnp.float32),
            pltpu.VMEM((K, D), jnp.float32),
            pltpu.VMEM_SHARED((AGG_ROWS, D), jnp.float32),
            pltpu.SemaphoreType.DMA,
        ],
    )
    parts = deg_call(dst3)

    g0, dinvb = pl.pallas_call(
        _prologue_tc,
        out_shape=[jax.ShapeDtypeStruct((N, D), jnp.float32)] * 2,
    )(parts, x)

    agg_call = pl.kernel(
        _agg_body,
        out_type=jax.ShapeDtypeStruct((NC, N, D), jnp.float32),
        mesh=mesh,
        scratch_types=[
            pltpu.VMEM((SS, K), jnp.int32),
            pltpu.VMEM((SS, K), jnp.int32),
            pltpu.VMEM((NB, K, D), jnp.float32),
            pltpu.VMEM_SHARED((AGG_ROWS, D), jnp.float32),
            pltpu.SemaphoreType.DMA,
            pltpu.SemaphoreType.DMA,
            pltpu.SemaphoreType.DMA,
            pltpu.SemaphoreType.DMA,
            pltpu.SemaphoreType.DMA,
            pltpu.SemaphoreType.DMA,
            pltpu.SemaphoreType.DMA,
            pltpu.SemaphoreType.DMA,
        ],
    )

    mm_call = pl.pallas_call(
        _mm_tc,
        grid=(N // BN,),
        in_specs=[pl.BlockSpec((BN, D), lambda i: (i, 0))] * 3 + [
            pl.BlockSpec((D, D), lambda i: (0, 0)),
            pl.BlockSpec((1, D), lambda i: (0, 0)),
        ],
        out_specs=[pl.BlockSpec((BN, D), lambda i: (i, 0))] * 2,
        out_shape=[jax.ShapeDtypeStruct((N, D), jnp.float32)] * 2,
    )

    g = g0
    outs = []
    for i in range(num_layers):
        r = agg_call(g, src2, dst2)
        h, g = mm_call(r[0], r[1], dinvb, W[i], b[i].reshape(1, D))
        outs.append(h)
    return jnp.concatenate(outs, axis=1)


# speedup vs baseline: 2.7363x; 1.2738x over previous
import jax
import jax.numpy as jnp
from jax import lax
from jax.experimental import pallas as pl
from jax.experimental.pallas import tpu as pltpu
from jax.experimental.pallas import tpu_sc as plsc

N = 10000
E = 320000
D = 128
NC = 2
NS = 16
NW = NC * NS
K = 64
CH = 160
EPW = K * CH
E_PAD = NW * EPW
STRIP = 640
AGG_ROWS = NS * STRIP
BN = 2000
LAST = N - (NS - 1) * STRIP
NB = 4
SS = 40
STAGES0 = 4
STAGES1 = 4


def _deg_body(dst_hbm, out_hbm, dst_v, ones_v, zero_v, deg_sh, dsem):
    c = lax.axis_index("c")
    s = lax.axis_index("s")
    w = c * NS + s

    def fill_step(i, carry):
        for j in range(D // 16):
            ones_v[i, pl.ds(j * 16, 16)] = jnp.full((16,), 1.0, jnp.float32)
            zero_v[i, pl.ds(j * 16, 16)] = jnp.zeros((16,), jnp.float32)
        return carry

    lax.fori_loop(0, K, fill_step, 0)
    for k in range(STRIP // K):
        pltpu.sync_copy(zero_v, deg_sh.at[pl.ds(s * STRIP + k * K, K)])
    plsc.subcore_barrier()

    pltpu.sync_copy(dst_hbm.at[w], dst_v)

    for j in range(4):
        pltpu.async_copy(ones_v, deg_sh.at[dst_v.at[j]], dsem, add=True)

    def step(k, carry):
        @pl.when(k + 4 < CH)
        def _issue():
            pltpu.async_copy(ones_v, deg_sh.at[dst_v.at[k + 4]], dsem,
                             add=True)

        pltpu.make_async_copy(ones_v, deg_sh.at[dst_v.at[0]], dsem).wait()
        return carry

    lax.fori_loop(0, CH, step, 0)
    plsc.subcore_barrier()

    base = s * STRIP

    @pl.when(s < NS - 1)
    def _copy_full():
        pltpu.sync_copy(deg_sh.at[pl.ds(base, STRIP)],
                        out_hbm.at[c, pl.ds(base, STRIP)])

    @pl.when(s == NS - 1)
    def _copy_tail():
        pltpu.sync_copy(deg_sh.at[pl.ds(base, LAST)],
                        out_hbm.at[c, pl.ds(base, LAST)])


def _agg_pipeline(g_hbm, src_hbm, dst_hbm, src_v, dst_v, rows_v, agg_sh,
                  sems, base, nstages):
    for stage in range(nstages):
        pltpu.sync_copy(src_hbm.at[pl.ds(base + stage * SS, SS)], src_v)
        pltpu.sync_copy(dst_hbm.at[pl.ds(base + stage * SS, SS)], dst_v)

        for j in range(NB - 1):
            pltpu.async_copy(g_hbm.at[src_v.at[j]], rows_v.at[j], sems[j])

        def outer(i, carry):
            for b in range(NB):
                k = i * NB + b
                bj = (b + NB - 1) % NB
                pltpu.make_async_copy(g_hbm.at[src_v.at[k]], rows_v.at[b],
                                      sems[b]).wait()
                pltpu.async_copy(rows_v.at[b], agg_sh.at[dst_v.at[k]],
                                 sems[b], add=True)

                @pl.when(k > 0)
                def _wait_prev_scatter():
                    pltpu.make_async_copy(rows_v.at[bj],
                                          agg_sh.at[dst_v.at[k]],
                                          sems[bj]).wait()

                @pl.when(k + NB - 1 < SS)
                def _issue_next_gather():
                    pltpu.async_copy(g_hbm.at[src_v.at[k + NB - 1]],
                                     rows_v.at[bj], sems[bj])
            return carry

        lax.fori_loop(0, SS // NB, outer, 0)
        pltpu.make_async_copy(rows_v.at[(SS - 1) % NB], agg_sh.at[dst_v.at[0]],
                              sems[(SS - 1) % NB]).wait()


def _agg_body(g_hbm, src_hbm, dst_hbm, out_hbm, src_v, dst_v, rows_v, agg_sh,
              sem0, sem1, sem2, sem3, sem4, sem5, sem6, sem7):
    sems = [sem0, sem1, sem2, sem3, sem4, sem5, sem6, sem7]
    c = lax.axis_index("c")
    s = lax.axis_index("s")

    def zero_step(i, carry):
        for j in range(D // 16):
            rows_v[0, i, pl.ds(j * 16, 16)] = jnp.zeros((16,), jnp.float32)
        return carry

    lax.fori_loop(0, K, zero_step, 0)
    for k in range(STRIP // K):
        pltpu.sync_copy(rows_v.at[0], agg_sh.at[pl.ds(s * STRIP + k * K, K)])
    plsc.subcore_barrier()

    @pl.when(c == 0)
    def _core0():
        _agg_pipeline(g_hbm, src_hbm, dst_hbm, src_v, dst_v, rows_v, agg_sh,
                      sems, s * (STAGES0 * SS), STAGES0)

    @pl.when(c == 1)
    def _core1():
        _agg_pipeline(g_hbm, src_hbm, dst_hbm, src_v, dst_v, rows_v, agg_sh,
                      sems, NS * STAGES0 * SS + s * (STAGES1 * SS), STAGES1)

    plsc.subcore_barrier()

    base = s * STRIP

    @pl.when(s < NS - 1)
    def _copy_full():
        pltpu.sync_copy(agg_sh.at[pl.ds(base, STRIP)],
                        out_hbm.at[c, pl.ds(base, STRIP)])

    @pl.when(s == NS - 1)
    def _copy_tail():
        pltpu.sync_copy(agg_sh.at[pl.ds(base, LAST)],
                        out_hbm.at[c, pl.ds(base, LAST)])


def _prologue_tc(parts_ref, x_ref, g_ref, dinv_ref):
    deg = parts_ref[0] + parts_ref[1]
    dinv = lax.rsqrt(jnp.maximum(deg, 1.0))
    g_ref[...] = dinv * x_ref[...]
    dinv_ref[...] = dinv


def _mm_tc(r0_ref, r1_ref, dinv_ref, w_ref, b_ref, h_ref, g_ref):
    u = dinv_ref[...] * (r0_ref[...] + r1_ref[...])
    h = jnp.dot(u, w_ref[...], preferred_element_type=jnp.float32) + b_ref[...]
    h_ref[...] = h
    g_ref[...] = dinv_ref[...] * h


def kernel(x, edge_index, W, b):
    assert x.shape == (N, D) and edge_index.shape == (2, E)
    num_layers = W.shape[0]
    src = edge_index[0].astype(jnp.int32)
    dst = edge_index[1].astype(jnp.int32)
    pad = E_PAD - E
    pi = jnp.arange(pad, dtype=jnp.int32)
    src_p = jnp.concatenate([src, pi % N])
    dst_p = jnp.concatenate([dst, N + pi % (AGG_ROWS - N)])
    src2 = src_p.reshape(E_PAD // K, K)
    dst2 = dst_p.reshape(E_PAD // K, K)
    dst3 = dst_p.reshape(NW, CH, K)

    mesh = plsc.VectorSubcoreMesh(core_axis_name="c", subcore_axis_name="s")

    deg_call = pl.kernel(
        _deg_body,
        out_type=jax.ShapeDtypeStruct((NC, N, D), jnp.float32),
        mesh=mesh,
        scratch_types=[
            pltpu.VMEM((CH, K), jnp.int32),
            pltpu.VMEM((K, D), jnp.float32),
            pltpu.VMEM((K, D), jnp.float32),
            pltpu.VMEM_SHARED((AGG_ROWS, D), jnp.float32),
            pltpu.SemaphoreType.DMA,
        ],
    )
    parts = deg_call(dst3)

    g0, dinvb = pl.pallas_call(
        _prologue_tc,
        out_shape=[jax.ShapeDtypeStruct((N, D), jnp.float32)] * 2,
    )(parts, x)

    agg_call = pl.kernel(
        _agg_body,
        out_type=jax.ShapeDtypeStruct((NC, N, D), jnp.float32),
        mesh=mesh,
        scratch_types=[
            pltpu.VMEM((SS, K), jnp.int32),
            pltpu.VMEM((SS, K), jnp.int32),
            pltpu.VMEM((NB, K, D), jnp.float32),
            pltpu.VMEM_SHARED((AGG_ROWS, D), jnp.float32),
            pltpu.SemaphoreType.DMA,
            pltpu.SemaphoreType.DMA,
            pltpu.SemaphoreType.DMA,
            pltpu.SemaphoreType.DMA,
            pltpu.SemaphoreType.DMA,
            pltpu.SemaphoreType.DMA,
            pltpu.SemaphoreType.DMA,
            pltpu.SemaphoreType.DMA,
        ],
    )

    mm_call = pl.pallas_call(
        _mm_tc,
        grid=(N // BN,),
        in_specs=[pl.BlockSpec((BN, D), lambda i: (i, 0))] * 3 + [
            pl.BlockSpec((D, D), lambda i: (0, 0)),
            pl.BlockSpec((1, D), lambda i: (0, 0)),
        ],
        out_specs=[pl.BlockSpec((BN, D), lambda i: (i, 0))] * 2,
        out_shape=[jax.ShapeDtypeStruct((N, D), jnp.float32)] * 2,
    )

    g = g0
    outs = []
    for i in range(num_layers):
        r = agg_call(g, src2, dst2)
        h, g = mm_call(r[0], r[1], dinvb, W[i], b[i].reshape(1, D))
        outs.append(h)
    return jnp.concatenate(outs, axis=1)
